# Initial kernel scaffold; baseline (speedup 1.0000x reference)
import jax
import jax.numpy as jnp
from jax.experimental import pallas as pl


def kernel(x, directed_edges, edge_weights):
    raise NotImplementedError("write your pallas kernel here")


# speedup vs baseline: 30.1821x; 30.1821x over previous
import functools

import jax
import jax.numpy as jnp
from jax import lax
from jax.experimental import pallas as pl
from jax.experimental.pallas import tpu as pltpu
from jax.experimental.pallas import tpu_sc as plsc

_B, _N, _C, _E, _D = 4, 512, 512, 16384, 2
_NTILES = 16
_NCORES = 2
_EPT = _E // _NTILES
_SLICE = (_N * _N) // _NTILES
_F32 = jnp.float32


def _sc_build_m(edges_soa, weights_soa):
    mesh = plsc.VectorSubcoreMesh(core_axis_name="c", subcore_axis_name="s")

    @functools.partial(
        pl.kernel,
        mesh=mesh,
        out_type=jax.ShapeDtypeStruct((_B, _D, _N * _N), _F32),
        scratch_types=[
            pltpu.VMEM((_EPT,), jnp.int32),
            pltpu.VMEM((_EPT,), jnp.int32),
            pltpu.VMEM((8, 128), jnp.int32),
            pltpu.VMEM((_EPT,), _F32),
            pltpu.VMEM((_EPT,), _F32),
            pltpu.VMEM((_SLICE,), _F32),
            pltpu.VMEM_SHARED((_N * _N,), _F32),
            pltpu.VMEM_SHARED((_N * _N,), _F32),
            pltpu.VMEM_SHARED((_N * _N,), _F32),
            pltpu.VMEM_SHARED((_N * _N,), _F32),
        ],
    )
    def k(edges_hbm, w_hbm, m_hbm, t_v, s_v, idx_v, w0_v, w1_v, buf_v,
          m00, m01, m10, m11):
        cid = lax.axis_index("c")
        sid = lax.axis_index("s")

        def zbody(i, carry):
            buf_v[pl.ds(i * 16, 16)] = jnp.zeros((16,), _F32)
            return carry
        lax.fori_loop(0, _SLICE // 16, zbody, 0)
        for m in (m00, m01, m10, m11):
            pltpu.sync_copy(buf_v, m.at[pl.ds(sid * _SLICE, _SLICE)])
        plsc.subcore_barrier()

        for lb, md0, md1 in ((0, m00, m01), (1, m10, m11)):
            b = cid * 2 + lb
            pltpu.sync_copy(edges_hbm.at[b, 0, pl.ds(sid * _EPT, _EPT)], t_v)
            pltpu.sync_copy(edges_hbm.at[b, 1, pl.ds(sid * _EPT, _EPT)], s_v)
            pltpu.sync_copy(w_hbm.at[b, 0, pl.ds(sid * _EPT, _EPT)], w0_v)
            pltpu.sync_copy(w_hbm.at[b, 1, pl.ds(sid * _EPT, _EPT)], w1_v)
            for r in range(8):
                for kk in range(8):
                    off = (r * 8 + kk) * 16
                    tt = t_v[pl.ds(off, 16)]
                    ss = s_v[pl.ds(off, 16)]
                    idx_v[r, pl.ds(kk * 16, 16)] = tt * _N + ss
            for r in range(8):
                pltpu.sync_copy(w0_v.at[pl.ds(r * 128, 128)],
                                md0.at[idx_v.at[r]], add=True)
                pltpu.sync_copy(w1_v.at[pl.ds(r * 128, 128)],
                                md1.at[idx_v.at[r]], add=True)
        plsc.subcore_barrier()

        for lb, dd, m in ((0, 0, m00), (0, 1, m01), (1, 0, m10), (1, 1, m11)):
            b = cid * 2 + lb
            pltpu.sync_copy(m.at[pl.ds(sid * _SLICE, _SLICE)], buf_v)
            pltpu.sync_copy(buf_v, m_hbm.at[b, dd, pl.ds(sid * _SLICE, _SLICE)])

    return k(edges_soa, weights_soa)


def _tc_apply(m, x):
    hi = jax.lax.Precision.HIGHEST

    def body(m_ref, x_ref, o_ref):
        xb = x_ref[0]
        m0 = m_ref[0, 0]
        m1 = m_ref[0, 1]
        y0 = jax.lax.dot(m0, xb, precision=hi, preferred_element_type=_F32)
        y1 = jax.lax.dot(m1, xb, precision=hi, preferred_element_type=_F32)
        w0 = jnp.sum(m0, axis=1, keepdims=True)
        w1 = jnp.sum(m1, axis=1, keepdims=True)
        z0 = y0 - w0 * xb
        z1 = y1 - w1 * xb
        cc = jax.lax.broadcasted_iota(jnp.int32, (_C, _C * _D), 0)
        jj = jax.lax.broadcasted_iota(jnp.int32, (_C, _C * _D), 1)
        e0 = (jj == _D * cc).astype(_F32)
        e1 = (jj == _D * cc + 1).astype(_F32)
        o_ref[0] = (jax.lax.dot(z0, e0, precision=hi, preferred_element_type=_F32)
                    + jax.lax.dot(z1, e1, precision=hi, preferred_element_type=_F32))

    return pl.pallas_call(
        body,
        grid=(_B,),
        in_specs=[
            pl.BlockSpec((1, _D, _N, _N), lambda b: (b, 0, 0, 0)),
            pl.BlockSpec((1, _N, _C), lambda b: (b, 0, 0)),
        ],
        out_specs=pl.BlockSpec((1, _N, _C * _D), lambda b: (b, 0, 0)),
        out_shape=jax.ShapeDtypeStruct((_B, _N, _C * _D), _F32),
    )(m, x)


def kernel(x, directed_edges, edge_weights):
    edges_soa = jnp.transpose(directed_edges, (0, 2, 1))
    w_soa = jnp.transpose(edge_weights, (0, 2, 1))
    m = _sc_build_m(edges_soa, w_soa)
    return _tc_apply(m.reshape(_B, _D, _N, _N), x)


# speedup vs baseline: 39.6393x; 1.3133x over previous
import functools

import jax
import jax.numpy as jnp
from jax import lax
from jax.experimental import pallas as pl
from jax.experimental.pallas import tpu as pltpu
from jax.experimental.pallas import tpu_sc as plsc

_B, _N, _C, _E, _D = 4, 512, 512, 16384, 2
_NTILES = 16
_NCORES = 2
_EPT = _E // _NTILES
_SLICE = (_N * _N) // _NTILES
_F32 = jnp.float32


def _sc_build_m(edges_soa, weights_soa):
    mesh = plsc.VectorSubcoreMesh(core_axis_name="c", subcore_axis_name="s")

    @functools.partial(
        pl.kernel,
        mesh=mesh,
        out_type=jax.ShapeDtypeStruct((_B, _D, _N * _N), _F32),
        scratch_types=[
            pltpu.VMEM((_EPT,), jnp.int32),
            pltpu.VMEM((_EPT,), jnp.int32),
            pltpu.VMEM((8, 128), jnp.int32),
            pltpu.VMEM((_EPT,), _F32),
            pltpu.VMEM((_EPT,), _F32),
            pltpu.VMEM((_SLICE,), _F32),
            pltpu.VMEM_SHARED((_N * _N,), _F32),
            pltpu.VMEM_SHARED((_N * _N,), _F32),
            pltpu.VMEM_SHARED((_N * _N,), _F32),
            pltpu.VMEM_SHARED((_N * _N,), _F32),
        ],
    )
    def k(edges_hbm, w_hbm, m_hbm, t_v, s_v, idx_v, w0_v, w1_v, buf_v,
          m00, m01, m10, m11):
        cid = lax.axis_index("c")
        sid = lax.axis_index("s")

        def zbody(i, carry):
            buf_v[pl.ds(i * 16, 16)] = jnp.zeros((16,), _F32)
            return carry
        lax.fori_loop(0, _SLICE // 16, zbody, 0)
        for m in (m00, m01, m10, m11):
            pltpu.sync_copy(buf_v, m.at[pl.ds(sid * _SLICE, _SLICE)])
        plsc.subcore_barrier()

        for lb, md0, md1 in ((0, m00, m01), (1, m10, m11)):
            b = cid * 2 + lb
            pltpu.sync_copy(edges_hbm.at[b, 0, pl.ds(sid * _EPT, _EPT)], t_v)
            pltpu.sync_copy(edges_hbm.at[b, 1, pl.ds(sid * _EPT, _EPT)], s_v)
            pltpu.sync_copy(w_hbm.at[b, 0, pl.ds(sid * _EPT, _EPT)], w0_v)
            pltpu.sync_copy(w_hbm.at[b, 1, pl.ds(sid * _EPT, _EPT)], w1_v)
            for r in range(8):
                for kk in range(8):
                    off = (r * 8 + kk) * 16
                    tt = t_v[pl.ds(off, 16)]
                    ss = s_v[pl.ds(off, 16)]
                    idx_v[r, pl.ds(kk * 16, 16)] = tt * _N + ss
            for r in range(8):
                pltpu.sync_copy(w0_v.at[pl.ds(r * 128, 128)],
                                md0.at[idx_v.at[r]], add=True)
                pltpu.sync_copy(w1_v.at[pl.ds(r * 128, 128)],
                                md1.at[idx_v.at[r]], add=True)
        plsc.subcore_barrier()

        for lb, dd, m in ((0, 0, m00), (0, 1, m01), (1, 0, m10), (1, 1, m11)):
            b = cid * 2 + lb
            pltpu.sync_copy(m.at[pl.ds(sid * _SLICE, _SLICE)], buf_v)
            pltpu.sync_copy(buf_v, m_hbm.at[b, dd, pl.ds(sid * _SLICE, _SLICE)])

    return k(edges_soa, weights_soa)


def _tc_apply(m, x):
    def body(m_ref, m16_ref, x_ref, x16_ref, p_ref, o_ref):
        xb = x_ref[0]
        xb16 = x16_ref[0]
        m0 = m_ref[0, 0]
        m1 = m_ref[0, 1]
        y0 = jax.lax.dot(m16_ref[0, 0], xb16, preferred_element_type=_F32)
        y1 = jax.lax.dot(m16_ref[0, 1], xb16, preferred_element_type=_F32)
        w0 = jnp.sum(m0, axis=1, keepdims=True)
        w1 = jnp.sum(m1, axis=1, keepdims=True)
        zc = jnp.concatenate([y0 - w0 * xb, y1 - w1 * xb],
                             axis=-1).astype(jnp.bfloat16)
        o_ref[0] = jax.lax.dot(zc, p_ref[...], preferred_element_type=_F32)

    j = jnp.arange(_C * _D)
    kk = jnp.arange(_C * _D)[:, None]
    perm = (kk == (j % _D) * _C + j // _D).astype(jnp.bfloat16)

    return pl.pallas_call(
        body,
        grid=(_B,),
        in_specs=[
            pl.BlockSpec((1, _D, _N, _N), lambda b: (b, 0, 0, 0)),
            pl.BlockSpec((1, _D, _N, _N), lambda b: (b, 0, 0, 0)),
            pl.BlockSpec((1, _N, _C), lambda b: (b, 0, 0)),
            pl.BlockSpec((1, _N, _C), lambda b: (b, 0, 0)),
            pl.BlockSpec((_C * _D, _C * _D), lambda b: (0, 0)),
        ],
        out_specs=pl.BlockSpec((1, _N, _C * _D), lambda b: (b, 0, 0)),
        out_shape=jax.ShapeDtypeStruct((_B, _N, _C * _D), _F32),
    )(m, m.astype(jnp.bfloat16), x, x.astype(jnp.bfloat16), perm)


def kernel(x, directed_edges, edge_weights):
    edges_soa = jnp.transpose(directed_edges, (0, 2, 1))
    w_soa = jnp.transpose(edge_weights, (0, 2, 1))
    m = _sc_build_m(edges_soa, w_soa)
    return _tc_apply(m.reshape(_B, _D, _N, _N), x)


# speedup vs baseline: 43.3448x; 1.0935x over previous
import functools

import jax
import jax.numpy as jnp
from jax import lax
from jax.experimental import pallas as pl
from jax.experimental.pallas import tpu as pltpu
from jax.experimental.pallas import tpu_sc as plsc

_B, _N, _C, _E, _D = 4, 512, 512, 16384, 2
_NTILES = 16
_NCORES = 2
_EPT = _E // _NTILES
_SLICE = (_N * _N) // _NTILES
_F32 = jnp.float32


def _sc_build_m(edges_soa, weights_soa):
    mesh = plsc.VectorSubcoreMesh(core_axis_name="c", subcore_axis_name="s")

    @functools.partial(
        pl.kernel,
        mesh=mesh,
        out_type=jax.ShapeDtypeStruct((_B, _D, _N * _N), _F32),
        scratch_types=[
            pltpu.VMEM((_EPT,), jnp.int32),
            pltpu.VMEM((_EPT,), jnp.int32),
            pltpu.VMEM((8, 128), jnp.int32),
            pltpu.VMEM((_EPT,), _F32),
            pltpu.VMEM((_EPT,), _F32),
            pltpu.VMEM((_SLICE,), _F32),
            pltpu.VMEM_SHARED((_N * _N,), _F32),
            pltpu.VMEM_SHARED((_N * _N,), _F32),
            pltpu.VMEM_SHARED((_N * _N,), _F32),
            pltpu.VMEM_SHARED((_N * _N,), _F32),
        ],
    )
    def k(edges_hbm, w_hbm, m_hbm, t_v, s_v, idx_v, w0_v, w1_v, buf_v,
          m00, m01, m10, m11):
        cid = lax.axis_index("c")
        sid = lax.axis_index("s")

        def zbody(i, carry):
            buf_v[pl.ds(i * 16, 16)] = jnp.zeros((16,), _F32)
            return carry
        lax.fori_loop(0, _SLICE // 16, zbody, 0)
        for m in (m00, m01, m10, m11):
            pltpu.sync_copy(buf_v, m.at[pl.ds(sid * _SLICE, _SLICE)])
        plsc.subcore_barrier()

        for lb, md0, md1 in ((0, m00, m01), (1, m10, m11)):
            b = cid * 2 + lb
            pltpu.sync_copy(edges_hbm.at[b, 0, pl.ds(sid * _EPT, _EPT)], t_v)
            pltpu.sync_copy(edges_hbm.at[b, 1, pl.ds(sid * _EPT, _EPT)], s_v)
            pltpu.sync_copy(w_hbm.at[b, 0, pl.ds(sid * _EPT, _EPT)], w0_v)
            pltpu.sync_copy(w_hbm.at[b, 1, pl.ds(sid * _EPT, _EPT)], w1_v)
            for r in range(8):
                for kk in range(8):
                    off = (r * 8 + kk) * 16
                    tt = t_v[pl.ds(off, 16)]
                    ss = s_v[pl.ds(off, 16)]
                    idx_v[r, pl.ds(kk * 16, 16)] = tt * _N + ss
            for r in range(8):
                pltpu.sync_copy(w0_v.at[pl.ds(r * 128, 128)],
                                md0.at[idx_v.at[r]], add=True)
                pltpu.sync_copy(w1_v.at[pl.ds(r * 128, 128)],
                                md1.at[idx_v.at[r]], add=True)
        plsc.subcore_barrier()

        for lb, dd, m in ((0, 0, m00), (0, 1, m01), (1, 0, m10), (1, 1, m11)):
            b = cid * 2 + lb
            pltpu.sync_copy(m.at[pl.ds(sid * _SLICE, _SLICE)], buf_v)
            pltpu.sync_copy(buf_v, m_hbm.at[b, dd, pl.ds(sid * _SLICE, _SLICE)])

    return k(edges_soa, weights_soa)


def _tc_apply(m, x):
    def body(m_ref, x_ref, p_ref, o_ref):
        xb = x_ref[0]
        xb16 = xb.astype(jnp.bfloat16)
        m0 = m_ref[0, 0]
        m1 = m_ref[0, 1]
        y0 = jax.lax.dot(m0.astype(jnp.bfloat16), xb16,
                         preferred_element_type=_F32)
        y1 = jax.lax.dot(m1.astype(jnp.bfloat16), xb16,
                         preferred_element_type=_F32)
        w0 = jnp.sum(m0, axis=1, keepdims=True)
        w1 = jnp.sum(m1, axis=1, keepdims=True)
        zc = jnp.concatenate([y0 - w0 * xb, y1 - w1 * xb],
                             axis=-1).astype(jnp.bfloat16)
        o_ref[0] = jax.lax.dot(zc, p_ref[...], preferred_element_type=_F32)

    j = jnp.arange(_C * _D)
    kk = jnp.arange(_C * _D)[:, None]
    perm = (kk == (j % _D) * _C + j // _D).astype(jnp.bfloat16)

    return pl.pallas_call(
        body,
        grid=(_B,),
        in_specs=[
            pl.BlockSpec((1, _D, _N, _N), lambda b: (b, 0, 0, 0)),
            pl.BlockSpec((1, _N, _C), lambda b: (b, 0, 0)),
            pl.BlockSpec((_C * _D, _C * _D), lambda b: (0, 0)),
        ],
        out_specs=pl.BlockSpec((1, _N, _C * _D), lambda b: (b, 0, 0)),
        out_shape=jax.ShapeDtypeStruct((_B, _N, _C * _D), _F32),
    )(m, x, perm)


def kernel(x, directed_edges, edge_weights):
    edges_soa = jnp.transpose(directed_edges, (0, 2, 1))
    w_soa = jnp.transpose(edge_weights, (0, 2, 1))
    m = _sc_build_m(edges_soa, w_soa)
    return _tc_apply(m.reshape(_B, _D, _N, _N), x)


# speedup vs baseline: 43.4612x; 1.0027x over previous
import functools

import jax
import jax.numpy as jnp
from jax import lax
from jax.experimental import pallas as pl
from jax.experimental.pallas import tpu as pltpu
from jax.experimental.pallas import tpu_sc as plsc

_B, _N, _C, _E, _D = 4, 512, 512, 16384, 2
_NTILES = 16
_NCORES = 2
_EPT = _E // _NTILES
_SLICE = (_N * _N) // _NTILES
_F32 = jnp.float32


def _sc_build_m(edges_soa, weights_soa):
    mesh = plsc.VectorSubcoreMesh(core_axis_name="c", subcore_axis_name="s")

    @functools.partial(
        pl.kernel,
        mesh=mesh,
        out_type=jax.ShapeDtypeStruct((_B, _D, _N * _N), _F32),
        scratch_types=[
            pltpu.VMEM((_EPT,), jnp.int32),
            pltpu.VMEM((_EPT,), jnp.int32),
            pltpu.VMEM((8, 128), jnp.int32),
            pltpu.VMEM((_EPT,), _F32),
            pltpu.VMEM((_EPT,), _F32),
            pltpu.VMEM((_SLICE,), _F32),
            pltpu.VMEM_SHARED((_N * _N,), _F32),
            pltpu.VMEM_SHARED((_N * _N,), _F32),
            pltpu.VMEM_SHARED((_N * _N,), _F32),
            pltpu.VMEM_SHARED((_N * _N,), _F32),
        ],
    )
    def k(edges_hbm, w_hbm, m_hbm, t_v, s_v, idx_v, w0_v, w1_v, buf_v,
          m00, m01, m10, m11):
        cid = lax.axis_index("c")
        sid = lax.axis_index("s")

        def zbody(i, carry):
            buf_v[pl.ds(i * 16, 16)] = jnp.zeros((16,), _F32)
            return carry
        lax.fori_loop(0, _SLICE // 16, zbody, 0)
        for m in (m00, m01, m10, m11):
            pltpu.sync_copy(buf_v, m.at[pl.ds(sid * _SLICE, _SLICE)])
        plsc.subcore_barrier()

        for lb, md0, md1 in ((0, m00, m01), (1, m10, m11)):
            b = cid * 2 + lb
            pltpu.sync_copy(edges_hbm.at[b, 0, pl.ds(sid * _EPT, _EPT)], t_v)
            pltpu.sync_copy(edges_hbm.at[b, 1, pl.ds(sid * _EPT, _EPT)], s_v)
            pltpu.sync_copy(w_hbm.at[b, 0, pl.ds(sid * _EPT, _EPT)], w0_v)
            pltpu.sync_copy(w_hbm.at[b, 1, pl.ds(sid * _EPT, _EPT)], w1_v)
            for r in range(8):
                for kk in range(8):
                    off = (r * 8 + kk) * 16
                    tt = t_v[pl.ds(off, 16)]
                    ss = s_v[pl.ds(off, 16)]
                    idx_v[r, pl.ds(kk * 16, 16)] = tt * _N + ss
            for r in range(8):
                pltpu.sync_copy(w0_v.at[pl.ds(r * 128, 128)],
                                md0.at[idx_v.at[r]], add=True)
                pltpu.sync_copy(w1_v.at[pl.ds(r * 128, 128)],
                                md1.at[idx_v.at[r]], add=True)
        plsc.subcore_barrier()

        for lb, dd, m in ((0, 0, m00), (0, 1, m01), (1, 0, m10), (1, 1, m11)):
            b = cid * 2 + lb
            pltpu.sync_copy(m.at[pl.ds(sid * _SLICE, _SLICE)],
                            m_hbm.at[b, dd, pl.ds(sid * _SLICE, _SLICE)])

    return k(edges_soa, weights_soa)


def _tc_apply(m, x):
    def body(m_ref, x_ref, p_ref, o_ref):
        xb = x_ref[0]
        xb16 = xb.astype(jnp.bfloat16)
        m0 = m_ref[0, 0]
        m1 = m_ref[0, 1]
        y0 = jax.lax.dot(m0.astype(jnp.bfloat16), xb16,
                         preferred_element_type=_F32)
        y1 = jax.lax.dot(m1.astype(jnp.bfloat16), xb16,
                         preferred_element_type=_F32)
        w0 = jnp.sum(m0, axis=1, keepdims=True)
        w1 = jnp.sum(m1, axis=1, keepdims=True)
        zc = jnp.concatenate([y0 - w0 * xb, y1 - w1 * xb],
                             axis=-1).astype(jnp.bfloat16)
        o_ref[0] = jax.lax.dot(zc, p_ref[...], preferred_element_type=_F32)

    j = jnp.arange(_C * _D)
    kk = jnp.arange(_C * _D)[:, None]
    perm = (kk == (j % _D) * _C + j // _D).astype(jnp.bfloat16)

    return pl.pallas_call(
        body,
        grid=(_B,),
        in_specs=[
            pl.BlockSpec((1, _D, _N, _N), lambda b: (b, 0, 0, 0)),
            pl.BlockSpec((1, _N, _C), lambda b: (b, 0, 0)),
            pl.BlockSpec((_C * _D, _C * _D), lambda b: (0, 0)),
        ],
        out_specs=pl.BlockSpec((1, _N, _C * _D), lambda b: (b, 0, 0)),
        out_shape=jax.ShapeDtypeStruct((_B, _N, _C * _D), _F32),
    )(m, x, perm)


def kernel(x, directed_edges, edge_weights):
    edges_soa = jnp.transpose(directed_edges, (0, 2, 1))
    w_soa = jnp.transpose(edge_weights, (0, 2, 1))
    m = _sc_build_m(edges_soa, w_soa)
    return _tc_apply(m.reshape(_B, _D, _N, _N), x)


# speedup vs baseline: 48.8645x; 1.1243x over previous
import functools

import jax
import jax.numpy as jnp
from jax import lax
from jax.experimental import pallas as pl
from jax.experimental.pallas import tpu as pltpu
from jax.experimental.pallas import tpu_sc as plsc

_B, _N, _C, _E, _D = 4, 512, 512, 16384, 2
_NTILES = 16
_NCORES = 2
_EPT = _E // _NTILES
_SLICE = (_N * _N) // _NTILES
_F32 = jnp.float32


def _sc_build_m(edges_soa, weights_soa):
    mesh = plsc.VectorSubcoreMesh(core_axis_name="c", subcore_axis_name="s")

    @functools.partial(
        pl.kernel,
        mesh=mesh,
        out_type=jax.ShapeDtypeStruct((_B * _D, _N * _N), _F32),
        scratch_types=[
            pltpu.VMEM((_EPT,), jnp.int32),
            pltpu.VMEM((_EPT,), jnp.int32),
            pltpu.VMEM((8, 128), jnp.int32),
            pltpu.VMEM((_EPT,), _F32),
            pltpu.VMEM((_EPT,), _F32),
            pltpu.VMEM((_SLICE,), _F32),
            pltpu.VMEM_SHARED((_N * _N,), _F32),
            pltpu.VMEM_SHARED((_N * _N,), _F32),
            pltpu.VMEM_SHARED((_N * _N,), _F32),
            pltpu.VMEM_SHARED((_N * _N,), _F32),
        ],
    )
    def k(edges_hbm, w_hbm, m_hbm, t_v, s_v, idx_v, w0_v, w1_v, buf_v,
          m00, m01, m10, m11):
        cid = lax.axis_index("c")
        sid = lax.axis_index("s")

        def zbody(i, carry):
            buf_v[pl.ds(i * 16, 16)] = jnp.zeros((16,), _F32)
            return carry
        lax.fori_loop(0, _SLICE // 16, zbody, 0)
        for m in (m00, m01, m10, m11):
            pltpu.sync_copy(buf_v, m.at[pl.ds(sid * _SLICE, _SLICE)])
        plsc.subcore_barrier()

        for lb, md0, md1 in ((0, m00, m01), (1, m10, m11)):
            b = cid * 2 + lb
            pltpu.sync_copy(edges_hbm.at[b, 0, pl.ds(sid * _EPT, _EPT)], t_v)
            pltpu.sync_copy(edges_hbm.at[b, 1, pl.ds(sid * _EPT, _EPT)], s_v)
            pltpu.sync_copy(w_hbm.at[b, 0, pl.ds(sid * _EPT, _EPT)], w0_v)
            pltpu.sync_copy(w_hbm.at[b, 1, pl.ds(sid * _EPT, _EPT)], w1_v)
            for r in range(8):
                for kk in range(8):
                    off = (r * 8 + kk) * 16
                    tt = t_v[pl.ds(off, 16)]
                    ss = s_v[pl.ds(off, 16)]
                    idx_v[r, pl.ds(kk * 16, 16)] = tt * _N + ss
            for r in range(8):
                pltpu.sync_copy(w0_v.at[pl.ds(r * 128, 128)],
                                md0.at[idx_v.at[r]], add=True)
                pltpu.sync_copy(w1_v.at[pl.ds(r * 128, 128)],
                                md1.at[idx_v.at[r]], add=True)
        plsc.subcore_barrier()

        for lb, dd, m in ((0, 0, m00), (0, 1, m01), (1, 0, m10), (1, 1, m11)):
            b = cid * 2 + lb
            pltpu.sync_copy(m.at[pl.ds(sid * _SLICE, _SLICE)],
                            m_hbm.at[b * _D + dd, pl.ds(sid * _SLICE, _SLICE)])

    return k(edges_soa, weights_soa)


def _tc_apply(m, x):
    def body(m_ref, x_ref, p_ref, o_ref):
        xb = x_ref[0]
        xb16 = xb.astype(jnp.bfloat16)
        m0 = m_ref[0, 0]
        m1 = m_ref[0, 1]
        y0 = jax.lax.dot(m0.astype(jnp.bfloat16), xb16,
                         preferred_element_type=_F32)
        y1 = jax.lax.dot(m1.astype(jnp.bfloat16), xb16,
                         preferred_element_type=_F32)
        w0 = jnp.sum(m0, axis=1, keepdims=True)
        w1 = jnp.sum(m1, axis=1, keepdims=True)
        zc = jnp.concatenate([y0 - w0 * xb, y1 - w1 * xb],
                             axis=-1).astype(jnp.bfloat16)
        o_ref[0] = jax.lax.dot(zc, p_ref[...], preferred_element_type=_F32)

    j = jnp.arange(_C * _D)
    kk = jnp.arange(_C * _D)[:, None]
    perm = (kk == (j % _D) * _C + j // _D).astype(jnp.bfloat16)

    return pl.pallas_call(
        body,
        grid=(_B,),
        in_specs=[
            pl.BlockSpec((1, _D, _N, _N), lambda b: (b, 0, 0, 0)),
            pl.BlockSpec((1, _N, _C), lambda b: (b, 0, 0)),
            pl.BlockSpec((_C * _D, _C * _D), lambda b: (0, 0)),
        ],
        out_specs=pl.BlockSpec((1, _N, _C * _D), lambda b: (b, 0, 0)),
        out_shape=jax.ShapeDtypeStruct((_B, _N, _C * _D), _F32),
    )(m, x, perm)


def kernel(x, directed_edges, edge_weights):
    edges_soa = jnp.transpose(directed_edges, (0, 2, 1))
    w_soa = jnp.transpose(edge_weights, (0, 2, 1))
    m = _sc_build_m(edges_soa, w_soa)
    return _tc_apply(m.reshape(_B, _D, _N, _N), x)


# speedup vs baseline: 48.9483x; 1.0017x over previous
import functools

import jax
import jax.numpy as jnp
from jax import lax
from jax.experimental import pallas as pl
from jax.experimental.pallas import tpu as pltpu
from jax.experimental.pallas import tpu_sc as plsc

_B, _N, _C, _E, _D = 4, 512, 512, 16384, 2
_NTILES = 16
_NCORES = 2
_EPT = _E // _NTILES
_SLICE = (_N * _N) // _NTILES
_F32 = jnp.float32


def _sc_build_m(edges_soa, weights_soa):
    mesh = plsc.VectorSubcoreMesh(core_axis_name="c", subcore_axis_name="s")

    @functools.partial(
        pl.kernel,
        mesh=mesh,
        out_type=jax.ShapeDtypeStruct((_B * _D, _N * _N), _F32),
        scratch_types=[
            pltpu.VMEM((_EPT,), jnp.int32),
            pltpu.VMEM((_EPT,), jnp.int32),
            pltpu.VMEM((8, 128), jnp.int32),
            pltpu.VMEM((_EPT,), _F32),
            pltpu.VMEM((_EPT,), _F32),
            pltpu.VMEM((_SLICE,), _F32),
            pltpu.VMEM_SHARED((_N * _N,), _F32),
            pltpu.VMEM_SHARED((_N * _N,), _F32),
            pltpu.VMEM_SHARED((_N * _N,), _F32),
            pltpu.VMEM_SHARED((_N * _N,), _F32),
        ],
    )
    def k(edges_hbm, w_hbm, m_hbm, t_v, s_v, idx_v, w0_v, w1_v, buf_v,
          m00, m01, m10, m11):
        cid = lax.axis_index("c")
        sid = lax.axis_index("s")

        def zbody(i, carry):
            buf_v[pl.ds(i * 16, 16)] = jnp.zeros((16,), _F32)
            return carry
        lax.fori_loop(0, _SLICE // 16, zbody, 0)
        for m in (m00, m01, m10, m11):
            pltpu.sync_copy(buf_v, m.at[pl.ds(sid * _SLICE, _SLICE)])
        plsc.subcore_barrier()

        for lb, md0, md1 in ((0, m00, m01), (1, m10, m11)):
            b = cid * 2 + lb
            pltpu.sync_copy(edges_hbm.at[b, 0, pl.ds(sid * _EPT, _EPT)], t_v)
            pltpu.sync_copy(edges_hbm.at[b, 1, pl.ds(sid * _EPT, _EPT)], s_v)
            pltpu.sync_copy(w_hbm.at[b, 0, pl.ds(sid * _EPT, _EPT)], w0_v)
            pltpu.sync_copy(w_hbm.at[b, 1, pl.ds(sid * _EPT, _EPT)], w1_v)
            for r in range(8):
                for kk in range(8):
                    off = (r * 8 + kk) * 16
                    tt = t_v[pl.ds(off, 16)]
                    ss = s_v[pl.ds(off, 16)]
                    idx_v[r, pl.ds(kk * 16, 16)] = tt * _N + ss
            for r in range(8):
                pltpu.sync_copy(w0_v.at[pl.ds(r * 128, 128)],
                                md0.at[idx_v.at[r]], add=True)
                pltpu.sync_copy(w1_v.at[pl.ds(r * 128, 128)],
                                md1.at[idx_v.at[r]], add=True)
        plsc.subcore_barrier()

        for lb, dd, m in ((0, 0, m00), (0, 1, m01), (1, 0, m10), (1, 1, m11)):
            b = cid * 2 + lb
            pltpu.sync_copy(m.at[pl.ds(sid * _SLICE, _SLICE)],
                            m_hbm.at[b * _D + dd, pl.ds(sid * _SLICE, _SLICE)])

    return k(edges_soa, weights_soa)


def _tc_apply(m, x):
    def body(m_ref, x_ref, p_ref, o_ref):
        xb = x_ref[0]
        xb16 = xb.astype(jnp.bfloat16)
        m0 = m_ref[0, 0]
        m1 = m_ref[0, 1]
        y0 = jax.lax.dot(m0.astype(jnp.bfloat16), xb16,
                         preferred_element_type=_F32)
        y1 = jax.lax.dot(m1.astype(jnp.bfloat16), xb16,
                         preferred_element_type=_F32)
        w0 = jnp.sum(m0, axis=1, keepdims=True)
        w1 = jnp.sum(m1, axis=1, keepdims=True)
        zc = jnp.concatenate([y0 - w0 * xb, y1 - w1 * xb],
                             axis=-1).astype(jnp.bfloat16)
        o_ref[0] = jax.lax.dot(zc, p_ref[...], preferred_element_type=_F32)

    j = jnp.arange(_C * _D)
    kk = jnp.arange(_C * _D)[:, None]
    perm = (kk == (j % _D) * _C + j // _D).astype(jnp.bfloat16)

    return pl.pallas_call(
        body,
        grid=(_B,),
        in_specs=[
            pl.BlockSpec((1, _D, _N, _N), lambda b: (b, 0, 0, 0)),
            pl.BlockSpec((1, _N, _C), lambda b: (b, 0, 0)),
            pl.BlockSpec((_C * _D, _C * _D), lambda b: (0, 0)),
        ],
        out_specs=pl.BlockSpec((1, _N, _C * _D), lambda b: (b, 0, 0)),
        out_shape=jax.ShapeDtypeStruct((_B, _N, _C * _D), _F32),
    )(m, x, perm)


def kernel(x, directed_edges, edge_weights):
    edges_soa = jnp.transpose(directed_edges, (0, 2, 1))
    w_soa = jnp.transpose(edge_weights, (0, 2, 1))
    m = _sc_build_m(edges_soa, w_soa).reshape(_B, _D, _N, _N)
    return _tc_apply(m, x)


# speedup vs baseline: 52.3374x; 1.0692x over previous
import functools

import jax
import jax.numpy as jnp
from jax import lax
from jax.experimental import pallas as pl
from jax.experimental.pallas import tpu as pltpu
from jax.experimental.pallas import tpu_sc as plsc

_B, _N, _C, _E, _D = 4, 512, 512, 16384, 2
_NTILES = 16
_NCORES = 2
_EPT = _E // _NTILES
_SLICE = (_N * _N) // _NTILES
_F32 = jnp.float32


def _sc_build_m(edges_soa, weights_soa):
    mesh = plsc.VectorSubcoreMesh(core_axis_name="c", subcore_axis_name="s")

    @functools.partial(
        pl.kernel,
        mesh=mesh,
        out_type=jax.ShapeDtypeStruct((_B * _D * _N, _N), _F32),
        scratch_types=[
            pltpu.VMEM((_EPT,), jnp.int32),
            pltpu.VMEM((_EPT,), jnp.int32),
            pltpu.VMEM((8, 128), jnp.int32),
            pltpu.VMEM((_EPT,), _F32),
            pltpu.VMEM((_EPT,), _F32),
            pltpu.VMEM((_SLICE,), _F32),
            pltpu.VMEM_SHARED((_N * _N,), _F32),
            pltpu.VMEM_SHARED((_N * _N,), _F32),
            pltpu.VMEM_SHARED((_N * _N,), _F32),
            pltpu.VMEM_SHARED((_N * _N,), _F32),
            pltpu.SemaphoreType.DMA,
        ],
    )
    def k(edges_hbm, w_hbm, m_hbm, t_v, s_v, idx_v, w0_v, w1_v, buf_v,
          m00, m01, m10, m11, dma_sem):
        cid = lax.axis_index("c")
        sid = lax.axis_index("s")

        def zbody(i, carry):
            buf_v[pl.ds(i * 16, 16)] = jnp.zeros((16,), _F32)
            return carry
        lax.fori_loop(0, _SLICE // 16, zbody, 0)
        for m in (m00, m01, m10, m11):
            pltpu.sync_copy(buf_v, m.at[pl.ds(sid * _SLICE, _SLICE)])
        plsc.subcore_barrier()

        for lb, md0, md1 in ((0, m00, m01), (1, m10, m11)):
            b = cid * 2 + lb
            pltpu.sync_copy(edges_hbm.at[b, 0, pl.ds(sid * _EPT, _EPT)], t_v)
            pltpu.sync_copy(edges_hbm.at[b, 1, pl.ds(sid * _EPT, _EPT)], s_v)
            pltpu.sync_copy(w_hbm.at[b, 0, pl.ds(sid * _EPT, _EPT)], w0_v)
            pltpu.sync_copy(w_hbm.at[b, 1, pl.ds(sid * _EPT, _EPT)], w1_v)
            for r in range(8):
                for kk in range(8):
                    off = (r * 8 + kk) * 16
                    tt = t_v[pl.ds(off, 16)]
                    ss = s_v[pl.ds(off, 16)]
                    idx_v[r, pl.ds(kk * 16, 16)] = tt * _N + ss
            for r in range(8):
                pltpu.sync_copy(w0_v.at[pl.ds(r * 128, 128)],
                                md0.at[idx_v.at[r]], add=True)
                pltpu.sync_copy(w1_v.at[pl.ds(r * 128, 128)],
                                md1.at[idx_v.at[r]], add=True)
        plsc.subcore_barrier()

        rows = _SLICE // _N
        cps = []
        for lb, dd, m in ((0, 0, m00), (0, 1, m01), (1, 0, m10), (1, 1, m11)):
            b = cid * 2 + lb
            base = (b * _D + dd) * _N + sid * rows
            for j in range(rows):
                cp = pltpu.make_async_copy(
                    m.at[pl.ds((sid * rows + j) * _N, _N)],
                    m_hbm.at[base + j],
                    dma_sem)
                cp.start()
                cps.append(cp)
        for cp in cps:
            cp.wait()

    return k(edges_soa, weights_soa)


def _tc_apply(m, x):
    def body(m_ref, x_ref, p_ref, o_ref):
        xb = x_ref[0]
        xb16 = xb.astype(jnp.bfloat16)
        mm = m_ref[...]
        m0 = mm[:_N]
        m1 = mm[_N:]
        y0 = jax.lax.dot(m0.astype(jnp.bfloat16), xb16,
                         preferred_element_type=_F32)
        y1 = jax.lax.dot(m1.astype(jnp.bfloat16), xb16,
                         preferred_element_type=_F32)
        w0 = jnp.sum(m0, axis=1, keepdims=True)
        w1 = jnp.sum(m1, axis=1, keepdims=True)
        zc = jnp.concatenate([y0 - w0 * xb, y1 - w1 * xb],
                             axis=-1).astype(jnp.bfloat16)
        o_ref[0] = jax.lax.dot(zc, p_ref[...], preferred_element_type=_F32)

    j = jnp.arange(_C * _D)
    kk = jnp.arange(_C * _D)[:, None]
    perm = (kk == (j % _D) * _C + j // _D).astype(jnp.bfloat16)

    return pl.pallas_call(
        body,
        grid=(_B,),
        in_specs=[
            pl.BlockSpec((_D * _N, _N), lambda b: (b, 0)),
            pl.BlockSpec((1, _N, _C), lambda b: (b, 0, 0)),
            pl.BlockSpec((_C * _D, _C * _D), lambda b: (0, 0)),
        ],
        out_specs=pl.BlockSpec((1, _N, _C * _D), lambda b: (b, 0, 0)),
        out_shape=jax.ShapeDtypeStruct((_B, _N, _C * _D), _F32),
    )(m, x, perm)


def kernel(x, directed_edges, edge_weights):
    edges_soa = jnp.transpose(directed_edges, (0, 2, 1))
    w_soa = jnp.transpose(edge_weights, (0, 2, 1))
    m = _sc_build_m(edges_soa, w_soa)
    return _tc_apply(m, x)
